# baseline (speedup 1.0000x reference)
import functools

import jax
import jax.numpy as jnp
from jax import lax
from jax.experimental import pallas as pl
from jax.experimental.pallas import tpu as pltpu
from jax.experimental.pallas import tpu_sc as plsc

N = 10000
E = 160000
D = 128

NC = 2
NS = 16
NW = NC * NS
CHUNK = 128
NCHUNKS = E // CHUNK
GROUPS = CHUNK // 16
ROWS_A = 632
ROWS_LAST = N - 15 * ROWS_A

_MESH = plsc.VectorSubcoreMesh(core_axis_name="c", subcore_axis_name="s")
_SC_PARAMS = pltpu.CompilerParams(use_tc_tiling_on_sc=False)

_f32 = jnp.float32
_i32 = jnp.int32
_bf16 = jnp.bfloat16


def _sds(shape, dtype=_f32):
    return jax.ShapeDtypeStruct(shape, dtype)


_VCH = (NCHUNKS + NW - 1) // NW


@functools.partial(
    pl.kernel,
    out_type=(_sds((E, D), _bf16), _sds((E, D), _bf16),
              _sds((E, 16)), _sds((E, 16))),
    mesh=_MESH,
    scratch_types=[
        [pltpu.VMEM((CHUNK,), _i32)] * 2,
        [pltpu.VMEM((CHUNK,), _i32)] * 2,
        [pltpu.VMEM((CHUNK, D), _bf16)] * 2,
        [pltpu.VMEM((CHUNK, D), _bf16)] * 2,
        [pltpu.VMEM((CHUNK, 16), _f32)] * 2,
        [pltpu.VMEM((CHUNK, 16), _f32)] * 2,
        [pltpu.SemaphoreType.DMA] * 2,
        [pltpu.SemaphoreType.DMA] * 2,
        [pltpu.SemaphoreType.DMA] * 2,
    ],
    compiler_params=_SC_PARAMS,
)
def _assemble(gd_hbm, gs_hbm, p16_hbm, src_hbm, dst_hbm,
              gdg_hbm, gsg_hbm, posd_hbm, poss_hbm,
              idxd_v, idxs_v, bufa, bufb, bufpd, bufps, semi, semg, semw):
    wid = lax.axis_index("s") * NC + lax.axis_index("c")
    nch = (NCHUNKS - wid + NW - 1) // NW

    def fire_idx(j, p):
        base = (wid + j * NW) * CHUNK
        pltpu.async_copy(dst_hbm.at[pl.ds(base, CHUNK)], idxd_v[p], semi[p])
        pltpu.async_copy(src_hbm.at[pl.ds(base, CHUNK)], idxs_v[p], semi[p])

    def wait_idx(p):
        pltpu.make_async_copy(dst_hbm.at[pl.ds(0, CHUNK)], idxd_v[p],
                              semi[p]).wait()
        pltpu.make_async_copy(src_hbm.at[pl.ds(0, CHUNK)], idxs_v[p],
                              semi[p]).wait()

    def drain_writes(p):
        z = pl.ds(0, CHUNK)
        pltpu.make_async_copy(bufa[p], gdg_hbm.at[z], semw[p]).wait()
        pltpu.make_async_copy(bufb[p], gsg_hbm.at[z], semw[p]).wait()
        pltpu.make_async_copy(bufpd[p], posd_hbm.at[z], semw[p]).wait()
        pltpu.make_async_copy(bufps[p], poss_hbm.at[z], semw[p]).wait()

    def gathers_and_writes(j, p, fire_next_idx):
        cps = [
            pltpu.async_copy(gd_hbm.at[idxd_v[p]], bufa[p], semg[p]),
            pltpu.async_copy(gs_hbm.at[idxs_v[p]], bufb[p], semg[p]),
            pltpu.async_copy(p16_hbm.at[idxd_v[p]], bufpd[p], semg[p]),
            pltpu.async_copy(p16_hbm.at[idxs_v[p]], bufps[p], semg[p]),
        ]
        if fire_next_idx:
            fire_next_idx()
        for cp in cps:
            cp.wait()
        base = (wid + j * NW) * CHUNK
        pltpu.async_copy(bufa[p], gdg_hbm.at[pl.ds(base, CHUNK)], semw[p])
        pltpu.async_copy(bufb[p], gsg_hbm.at[pl.ds(base, CHUNK)], semw[p])
        pltpu.async_copy(bufpd[p], posd_hbm.at[pl.ds(base, CHUNK)], semw[p])
        pltpu.async_copy(bufps[p], poss_hbm.at[pl.ds(base, CHUNK)], semw[p])

    fire_idx(0, 0)

    def pair(i2, carry):
        for b in range(2):
            j = 2 * i2 + b
            p = b

            @pl.when(i2 >= 1)
            def _():
                drain_writes(p)

            wait_idx(p)
            gathers_and_writes(j, p, lambda: fire_idx(j + 1, 1 - p))
        return carry

    lax.fori_loop(0, (_VCH - 2) // 2, pair, 0)

    drain_writes(0)
    wait_idx(0)
    has39 = nch == _VCH

    def fire39():
        @pl.when(has39)
        def _():
            fire_idx(_VCH - 1, 1)

    gathers_and_writes(_VCH - 2, 0, fire39)
    drain_writes(1)

    @pl.when(has39)
    def _():
        wait_idx(1)
        gathers_and_writes(_VCH - 1, 1, None)
        drain_writes(1)

    drain_writes(0)


def _make_scatter(with_h):
    outs = []
    if with_h:
        outs += [_sds((N, D)), _sds((N, D))]
    outs += [_sds((N, 16)), _sds((N, 16))]

    scratch = [
        [pltpu.VMEM((CHUNK,), _i32)] * 2,
        [pltpu.VMEM((CHUNK, 16), _f32)] * 2,
        [pltpu.SemaphoreType.DMA] * 2,
    ]
    if with_h:
        scratch.insert(2, [pltpu.VMEM((CHUNK, D), _f32)] * 2)
        scratch += [pltpu.VMEM_SHARED((N, D), _f32)]
    scratch += [pltpu.VMEM_SHARED((N, 16), _f32)]

    def body(*refs):
        if with_h:
            (h_hbm, a_hbm, dst_hbm, z128_hbm, z16_hbm,
             sh0_hbm, sh1_hbm, sa0_hbm, sa1_hbm,
             idxd_v, abuf, hbuf,
             semin, sh_sp, sa_sp) = refs
        else:
            (a_hbm, dst_hbm, z16_hbm,
             sa0_hbm, sa1_hbm,
             idxd_v, abuf,
             semin, sa_sp) = refs

        cid = lax.axis_index("c")
        sid = lax.axis_index("s")
        wid = sid * NC + cid
        row0 = sid * ROWS_A

        @pl.when(sid < NS - 1)
        def _():
            pltpu.sync_copy(z16_hbm.at[pl.ds(row0, ROWS_A)],
                            sa_sp.at[pl.ds(row0, ROWS_A)])
            if with_h:
                pltpu.sync_copy(z128_hbm.at[pl.ds(row0, ROWS_A)],
                                sh_sp.at[pl.ds(row0, ROWS_A)])

        @pl.when(sid == NS - 1)
        def _():
            pltpu.sync_copy(z16_hbm.at[pl.ds(row0, ROWS_LAST)],
                            sa_sp.at[pl.ds(row0, ROWS_LAST)])
            if with_h:
                pltpu.sync_copy(z128_hbm.at[pl.ds(row0, ROWS_LAST)],
                                sh_sp.at[pl.ds(row0, ROWS_LAST)])
        plsc.subcore_barrier()

        nch = (NCHUNKS - wid + NW - 1) // NW

        def fire_in(j, p):
            base = (wid + j * NW) * CHUNK
            pltpu.async_copy(dst_hbm.at[pl.ds(base, CHUNK)], idxd_v[p],
                             semin[p])
            pltpu.async_copy(a_hbm.at[pl.ds(base, CHUNK)], abuf[p], semin[p])
            if with_h:
                pltpu.async_copy(h_hbm.at[pl.ds(base, CHUNK)], hbuf[p],
                                 semin[p])

        def wait_in(p):
            z = pl.ds(0, CHUNK)
            pltpu.make_async_copy(dst_hbm.at[z], idxd_v[p], semin[p]).wait()
            pltpu.make_async_copy(a_hbm.at[z], abuf[p], semin[p]).wait()
            if with_h:
                pltpu.make_async_copy(h_hbm.at[z], hbuf[p], semin[p]).wait()

        def scatter(p):
            pltpu.sync_copy(abuf[p], sa_sp.at[idxd_v[p]], add=True)
            if with_h:
                pltpu.sync_copy(hbuf[p], sh_sp.at[idxd_v[p]], add=True)

        fire_in(0, 0)

        def pair(i2, carry):
            for b in range(2):
                j = 2 * i2 + b
                wait_in(b)
                fire_in(j + 1, 1 - b)
                scatter(b)
            return carry

        lax.fori_loop(0, (_VCH - 2) // 2, pair, 0)
        has39 = nch == _VCH
        wait_in(0)

        @pl.when(has39)
        def _():
            fire_in(_VCH - 1, 1)

        scatter(0)

        @pl.when(has39)
        def _():
            wait_in(1)
            scatter(1)

        plsc.subcore_barrier()

        for nrows, is_last in ((ROWS_A, False), (ROWS_LAST, True)):
            cond = (sid == NS - 1) if is_last else (sid < NS - 1)
            rows = pl.ds(row0, nrows)

            @pl.when(jnp.logical_and(cond, cid == 0))
            def _(rows=rows):
                pltpu.sync_copy(sa_sp.at[rows], sa0_hbm.at[rows])
                if with_h:
                    pltpu.sync_copy(sh_sp.at[rows], sh0_hbm.at[rows])

            @pl.when(jnp.logical_and(cond, cid == 1))
            def _(rows=rows):
                pltpu.sync_copy(sa_sp.at[rows], sa1_hbm.at[rows])
                if with_h:
                    pltpu.sync_copy(sh_sp.at[rows], sh1_hbm.at[rows])

    return pl.kernel(body, out_type=tuple(outs), mesh=_MESH,
                     scratch_types=scratch, compiler_params=_SC_PARAMS)


_scatter_full = _make_scatter(True)
_scatter_aonly = _make_scatter(False)


_NB = 1000
_EB = 1280


def _full(shape):
    return pl.BlockSpec(shape, lambda i: (0,) * len(shape))


def _rows(shape):
    return pl.BlockSpec(shape, lambda i: (i,) + (0,) * (len(shape) - 1))


def _g0_body(x_ref, wa_ref, wb_ref, gd_ref, gs_ref):
    xb = x_ref[...]
    gd_ref[...] = jnp.dot(
        xb, wa_ref[...], preferred_element_type=_f32).astype(_bf16)
    gs_ref[...] = jnp.dot(
        xb, wb_ref[...], preferred_element_type=_f32).astype(_bf16)


def _g0(x, wa, wb):
    return pl.pallas_call(
        _g0_body,
        grid=(N // _NB,),
        in_specs=[_rows((_NB, D)), _full((D, D)), _full((D, D))],
        out_specs=(_rows((_NB, D)), _rows((_NB, D))),
        out_shape=(_sds((N, D), _bf16), _sds((N, D), _bf16)),
    )(x, wa, wb)


def _make_edge_body(with_h):
    def body(*refs):
        if with_h:
            (gdg, gsg, ea, posd, poss, w1c, b1, w2c, bc, vrow, a2b,
             h_ref, a16_ref) = refs
        else:
            (gdg, gsg, ea, posd, poss, w1c, b1, w2c, bc, vrow, a2b,
             a16_ref) = refs
        pre = (gdg[...].astype(_f32) + gsg[...].astype(_f32)
               + jnp.dot(ea[...], w1c[...], preferred_element_type=_f32)
               + b1[...])
        h = jnp.maximum(pre, 0.0)
        if with_h:
            h_ref[...] = h
        t = jnp.maximum(
            jnp.dot(h.astype(_bf16), w2c[...],
                    preferred_element_type=_f32) + bc[...], 0.0)
        wv = jnp.sum(t * vrow[...], axis=1, keepdims=True) + a2b[0, 0]
        d16 = poss[...] - posd[...]
        r2 = jnp.sum(d16 * d16, axis=1, keepdims=True)
        rinv = lax.rsqrt(jnp.maximum(r2, 1e-12))
        cols = lax.broadcasted_iota(_i32, (1, 16), 1)
        a16_ref[...] = d16 * (wv * rinv) + (cols == 3).astype(_f32)
    return body


def _edge(gdg, gsg, ea, posd, poss, w1c, b1, w2c, bc, vrow, a2b, with_h):
    in_specs = [
        _rows((_EB, D)), _rows((_EB, D)), _rows((_EB, D)),
        _rows((_EB, 16)), _rows((_EB, 16)),
        _full((D, D)), _full((1, D)), _full((D, 2 * D)), _full((1, 2 * D)),
        _full((1, 2 * D)), _full((1, 8)),
    ]
    if with_h:
        out_specs = (_rows((_EB, D)), _rows((_EB, 16)))
        out_shape = (_sds((E, D)), _sds((E, 16)))
    else:
        out_specs = (_rows((_EB, 16)),)
        out_shape = (_sds((E, 16)),)
    res = pl.pallas_call(
        _make_edge_body(with_h),
        grid=(E // _EB,),
        in_specs=in_specs,
        out_specs=out_specs,
        out_shape=out_shape,
    )(gdg, gsg, ea, posd, poss, w1c, b1, w2c, bc, vrow, a2b)
    return res if with_h else res[0]


def _node_body(x_ref, sh0, sh1, sa0, sa1, posp,
               n1a, w2n1b, cvec, nb1, n2, nb2, wan, wbn,
               xn_ref, gd_ref, gs_ref, pospn_ref):
    sa = sa0[...] + sa1[...]
    counts = sa[:, 3:4]
    inv = 1.0 / jnp.maximum(counts, 1.0)
    flag = jnp.minimum(counts, 1.0)
    cols = lax.broadcasted_iota(_i32, (1, 16), 1)
    mask = (cols < 3).astype(_f32)
    pospn_ref[...] = posp[...] + sa * inv * mask
    sh = sh0[...] + sh1[...]
    z = (jnp.dot(x_ref[...], n1a[...], preferred_element_type=_f32)
         + jnp.dot(sh, w2n1b[...], preferred_element_type=_f32) * inv
         + flag * cvec[...] + nb1[...])
    xn = jnp.dot(jnp.maximum(z, 0.0), n2[...],
                 preferred_element_type=_f32) + nb2[...]
    xn_ref[...] = xn
    gd_ref[...] = jnp.dot(
        xn, wan[...], preferred_element_type=_f32).astype(_bf16)
    gs_ref[...] = jnp.dot(
        xn, wbn[...], preferred_element_type=_f32).astype(_bf16)


def _node(x, sh0, sh1, sa0, sa1, posp, n1a, w2n1b, cvec, nb1, n2, nb2,
          wan, wbn):
    return pl.pallas_call(
        _node_body,
        grid=(N // _NB,),
        in_specs=[
            _rows((_NB, D)), _rows((_NB, D)), _rows((_NB, D)),
            _rows((_NB, 16)), _rows((_NB, 16)), _rows((_NB, 16)),
            _full((D, 2 * D)), _full((D, 2 * D)), _full((1, 2 * D)),
            _full((1, 2 * D)), _full((2 * D, D)), _full((1, D)),
            _full((D, D)), _full((D, D)),
        ],
        out_specs=(_rows((_NB, D)), _rows((_NB, D)), _rows((_NB, D)),
                   _rows((_NB, 16))),
        out_shape=(_sds((N, D)), _sds((N, D), _bf16), _sds((N, D), _bf16),
                   _sds((N, 16))),
    )(x, sh0, sh1, sa0, sa1, posp, n1a, w2n1b, cvec, nb1, n2, nb2, wan, wbn)


def _posup_body(sa0, sa1, posp, pospn_ref):
    sa = sa0[...] + sa1[...]
    counts = sa[:, 3:4]
    inv = 1.0 / jnp.maximum(counts, 1.0)
    cols = lax.broadcasted_iota(_i32, (1, 16), 1)
    mask = (cols < 3).astype(_f32)
    pospn_ref[...] = posp[...] + sa * inv * mask


def _posup(sa0, sa1, posp):
    return pl.pallas_call(
        _posup_body,
        grid=(N // _NB,),
        in_specs=[_rows((_NB, 16)), _rows((_NB, 16)), _rows((_NB, 16))],
        out_specs=_rows((_NB, 16)),
        out_shape=_sds((N, 16)),
    )(sa0, sa1, posp)


def kernel(x, edge_index, edge_attr, pos, params):
    srcv = edge_index[0].astype(_i32)
    dstv = edge_index[1].astype(_i32)
    posp = jnp.pad(pos, ((0, 0), (0, 13)))
    z128 = jnp.zeros((N, D), _f32)
    z16 = jnp.zeros((N, 16), _f32)

    folded = []
    for p in params:
        W1, b1 = p['msg']['W1'], p['msg']['b1']
        W2, b2 = p['msg']['W2'], p['msg']['b2']
        A1, a1b = p['accl']['W1'], p['accl']['b1']
        A2, a2b = p['accl']['W2'], p['accl']['b2']
        N1, nb1 = p['node']['W1'], p['node']['b1']
        N2, nb2 = p['node']['W2'], p['node']['b2']
        folded.append(dict(
            W1a=W1[:D], W1b=W1[D:2 * D], W1c=W1[2 * D:].astype(_bf16),
            b1=b1[None, :],
            W2c=(W2 @ A1).astype(_bf16), bc=(b2 @ A1 + a1b)[None, :],
            vrow=A2[:, 0][None, :],
            a2b=jnp.broadcast_to(a2b[0], (1, 8)),
            N1a=N1[:D], W2N1b=W2 @ N1[D:2 * D],
            cvec=(b2 @ N1[D:2 * D])[None, :], nb1=nb1[None, :],
            N2=N2, nb2=nb2[None, :],
        ))

    ea_bf = edge_attr.astype(_bf16)
    gd, gs = _g0(x, folded[0]['W1a'], folded[0]['W1b'])
    for l in range(3):
        f = folded[l]
        gdg, gsg, posd, poss = _assemble(gd, gs, posp, srcv, dstv)
        if l < 2:
            h, a16 = _edge(gdg, gsg, ea_bf, posd, poss, f['W1c'],
                           f['b1'], f['W2c'], f['bc'], f['vrow'], f['a2b'],
                           True)
            sh0, sh1, sa0, sa1 = _scatter_full(h, a16, dstv, z128, z16)
            fn = folded[l + 1]
            x, gd, gs, posp = _node(
                x, sh0, sh1, sa0, sa1, posp,
                f['N1a'], f['W2N1b'], f['cvec'], f['nb1'], f['N2'], f['nb2'],
                fn['W1a'], fn['W1b'])
        else:
            a16 = _edge(gdg, gsg, ea_bf, posd, poss, f['W1c'], f['b1'],
                        f['W2c'], f['bc'], f['vrow'], f['a2b'], False)
            sa0, sa1 = _scatter_aonly(a16, dstv, z16)
            posp = _posup(sa0, sa1, posp)
    return posp[:, :3]


# speedup vs baseline: 1.5039x; 1.5039x over previous
import functools

import jax
import jax.numpy as jnp
from jax import lax
from jax.experimental import pallas as pl
from jax.experimental.pallas import tpu as pltpu
from jax.experimental.pallas import tpu_sc as plsc

N = 10000
E = 160000
D = 128

NC = 2
NS = 16
NW = NC * NS
CHUNK = 128
NCHUNKS = E // CHUNK
GROUPS = CHUNK // 16
ROWS_A = 632
ROWS_LAST = N - 15 * ROWS_A

_MESH = plsc.VectorSubcoreMesh(core_axis_name="c", subcore_axis_name="s")
_SC_PARAMS = pltpu.CompilerParams(use_tc_tiling_on_sc=False)

_f32 = jnp.float32
_i32 = jnp.int32
_bf16 = jnp.bfloat16


def _sds(shape, dtype=_f32):
    return jax.ShapeDtypeStruct(shape, dtype)


_VCH = (NCHUNKS + NW - 1) // NW


@functools.partial(
    pl.kernel,
    out_type=(_sds((E, D)), _sds((E, D)), _sds((E, 16)), _sds((E, 16))),
    mesh=_MESH,
    scratch_types=[
        [pltpu.VMEM((CHUNK,), _i32)] * 2,
        [pltpu.VMEM((CHUNK,), _i32)] * 2,
        [pltpu.VMEM((CHUNK, D), _f32)] * 2,
        [pltpu.VMEM((CHUNK, D), _f32)] * 2,
        [pltpu.VMEM((CHUNK, 16), _f32)] * 2,
        [pltpu.VMEM((CHUNK, 16), _f32)] * 2,
        [pltpu.SemaphoreType.DMA] * 2,
        [pltpu.SemaphoreType.DMA] * 2,
        [pltpu.SemaphoreType.DMA] * 2,
    ],
    compiler_params=_SC_PARAMS,
)
def _assemble(gd_hbm, gs_hbm, p16_hbm, src_hbm, dst_hbm,
              gdg_hbm, gsg_hbm, posd_hbm, poss_hbm,
              idxd_v, idxs_v, bufa, bufb, bufpd, bufps, semi, semg, semw):
    wid = lax.axis_index("s") * NC + lax.axis_index("c")
    nch = (NCHUNKS - wid + NW - 1) // NW

    def fire_idx(j, p):
        base = (wid + j * NW) * CHUNK
        pltpu.async_copy(dst_hbm.at[pl.ds(base, CHUNK)], idxd_v[p], semi[p])
        pltpu.async_copy(src_hbm.at[pl.ds(base, CHUNK)], idxs_v[p], semi[p])

    def wait_idx(p):
        pltpu.make_async_copy(dst_hbm.at[pl.ds(0, CHUNK)], idxd_v[p],
                              semi[p]).wait()
        pltpu.make_async_copy(src_hbm.at[pl.ds(0, CHUNK)], idxs_v[p],
                              semi[p]).wait()

    def drain_writes(p):
        z = pl.ds(0, CHUNK)
        pltpu.make_async_copy(bufa[p], gdg_hbm.at[z], semw[p]).wait()
        pltpu.make_async_copy(bufb[p], gsg_hbm.at[z], semw[p]).wait()
        pltpu.make_async_copy(bufpd[p], posd_hbm.at[z], semw[p]).wait()
        pltpu.make_async_copy(bufps[p], poss_hbm.at[z], semw[p]).wait()

    def gathers_and_writes(j, p, fire_next_idx):
        cps = [
            pltpu.async_copy(gd_hbm.at[idxd_v[p]], bufa[p], semg[p]),
            pltpu.async_copy(gs_hbm.at[idxs_v[p]], bufb[p], semg[p]),
            pltpu.async_copy(p16_hbm.at[idxd_v[p]], bufpd[p], semg[p]),
            pltpu.async_copy(p16_hbm.at[idxs_v[p]], bufps[p], semg[p]),
        ]
        if fire_next_idx:
            fire_next_idx()
        for cp in cps:
            cp.wait()
        base = (wid + j * NW) * CHUNK
        pltpu.async_copy(bufa[p], gdg_hbm.at[pl.ds(base, CHUNK)], semw[p])
        pltpu.async_copy(bufb[p], gsg_hbm.at[pl.ds(base, CHUNK)], semw[p])
        pltpu.async_copy(bufpd[p], posd_hbm.at[pl.ds(base, CHUNK)], semw[p])
        pltpu.async_copy(bufps[p], poss_hbm.at[pl.ds(base, CHUNK)], semw[p])

    fire_idx(0, 0)

    def pair(i2, carry):
        for b in range(2):
            j = 2 * i2 + b
            p = b

            @pl.when(i2 >= 1)
            def _():
                drain_writes(p)

            wait_idx(p)
            gathers_and_writes(j, p, lambda: fire_idx(j + 1, 1 - p))
        return carry

    lax.fori_loop(0, (_VCH - 2) // 2, pair, 0)

    drain_writes(0)
    wait_idx(0)
    has39 = nch == _VCH

    def fire39():
        @pl.when(has39)
        def _():
            fire_idx(_VCH - 1, 1)

    gathers_and_writes(_VCH - 2, 0, fire39)
    drain_writes(1)

    @pl.when(has39)
    def _():
        wait_idx(1)
        gathers_and_writes(_VCH - 1, 1, None)
        drain_writes(1)

    drain_writes(0)


def _make_scatter(with_h):
    outs = []
    if with_h:
        outs += [_sds((N, D)), _sds((N, D))]
    outs += [_sds((N, 16)), _sds((N, 16))]

    scratch = [
        [pltpu.VMEM((CHUNK,), _i32)] * 2,
        [pltpu.VMEM((CHUNK, 16), _f32)] * 2,
        [pltpu.SemaphoreType.DMA] * 2,
    ]
    if with_h:
        scratch.insert(2, [pltpu.VMEM((CHUNK, D), _f32)] * 2)
        scratch += [pltpu.VMEM_SHARED((N, D), _f32)]
    scratch += [pltpu.VMEM_SHARED((N, 16), _f32)]

    def body(*refs):
        if with_h:
            (h_hbm, a_hbm, dst_hbm, z128_hbm, z16_hbm,
             sh0_hbm, sh1_hbm, sa0_hbm, sa1_hbm,
             idxd_v, abuf, hbuf,
             semin, sh_sp, sa_sp) = refs
        else:
            (a_hbm, dst_hbm, z16_hbm,
             sa0_hbm, sa1_hbm,
             idxd_v, abuf,
             semin, sa_sp) = refs

        cid = lax.axis_index("c")
        sid = lax.axis_index("s")
        wid = sid * NC + cid
        row0 = sid * ROWS_A

        @pl.when(sid < NS - 1)
        def _():
            pltpu.sync_copy(z16_hbm.at[pl.ds(row0, ROWS_A)],
                            sa_sp.at[pl.ds(row0, ROWS_A)])
            if with_h:
                pltpu.sync_copy(z128_hbm.at[pl.ds(row0, ROWS_A)],
                                sh_sp.at[pl.ds(row0, ROWS_A)])

        @pl.when(sid == NS - 1)
        def _():
            pltpu.sync_copy(z16_hbm.at[pl.ds(row0, ROWS_LAST)],
                            sa_sp.at[pl.ds(row0, ROWS_LAST)])
            if with_h:
                pltpu.sync_copy(z128_hbm.at[pl.ds(row0, ROWS_LAST)],
                                sh_sp.at[pl.ds(row0, ROWS_LAST)])
        plsc.subcore_barrier()

        nch = (NCHUNKS - wid + NW - 1) // NW

        def fire_in(j, p):
            base = (wid + j * NW) * CHUNK
            pltpu.async_copy(dst_hbm.at[pl.ds(base, CHUNK)], idxd_v[p],
                             semin[p])
            pltpu.async_copy(a_hbm.at[pl.ds(base, CHUNK)], abuf[p], semin[p])
            if with_h:
                pltpu.async_copy(h_hbm.at[pl.ds(base, CHUNK)], hbuf[p],
                                 semin[p])

        def wait_in(p):
            z = pl.ds(0, CHUNK)
            pltpu.make_async_copy(dst_hbm.at[z], idxd_v[p], semin[p]).wait()
            pltpu.make_async_copy(a_hbm.at[z], abuf[p], semin[p]).wait()
            if with_h:
                pltpu.make_async_copy(h_hbm.at[z], hbuf[p], semin[p]).wait()

        def scatter(p):
            pltpu.sync_copy(abuf[p], sa_sp.at[idxd_v[p]], add=True)
            if with_h:
                pltpu.sync_copy(hbuf[p], sh_sp.at[idxd_v[p]], add=True)

        fire_in(0, 0)

        def pair(i2, carry):
            for b in range(2):
                j = 2 * i2 + b
                wait_in(b)
                fire_in(j + 1, 1 - b)
                scatter(b)
            return carry

        lax.fori_loop(0, (_VCH - 2) // 2, pair, 0)
        has39 = nch == _VCH
        wait_in(0)

        @pl.when(has39)
        def _():
            fire_in(_VCH - 1, 1)

        scatter(0)

        @pl.when(has39)
        def _():
            wait_in(1)
            scatter(1)

        plsc.subcore_barrier()

        for nrows, is_last in ((ROWS_A, False), (ROWS_LAST, True)):
            cond = (sid == NS - 1) if is_last else (sid < NS - 1)
            rows = pl.ds(row0, nrows)

            @pl.when(jnp.logical_and(cond, cid == 0))
            def _(rows=rows):
                pltpu.sync_copy(sa_sp.at[rows], sa0_hbm.at[rows])
                if with_h:
                    pltpu.sync_copy(sh_sp.at[rows], sh0_hbm.at[rows])

            @pl.when(jnp.logical_and(cond, cid == 1))
            def _(rows=rows):
                pltpu.sync_copy(sa_sp.at[rows], sa1_hbm.at[rows])
                if with_h:
                    pltpu.sync_copy(sh_sp.at[rows], sh1_hbm.at[rows])

    return pl.kernel(body, out_type=tuple(outs), mesh=_MESH,
                     scratch_types=scratch, compiler_params=_SC_PARAMS)


_scatter_full = _make_scatter(True)
_scatter_aonly = _make_scatter(False)


_NB = 1000
_EB = 1280


def _full(shape):
    return pl.BlockSpec(shape, lambda i: (0,) * len(shape))


def _rows(shape):
    return pl.BlockSpec(shape, lambda i: (i,) + (0,) * (len(shape) - 1))


def _g0_body(x_ref, wa_ref, wb_ref, gd_ref, gs_ref):
    xb = x_ref[...]
    gd_ref[...] = jnp.dot(xb, wa_ref[...], preferred_element_type=_f32)
    gs_ref[...] = jnp.dot(xb, wb_ref[...], preferred_element_type=_f32)


def _g0(x, wa, wb):
    return pl.pallas_call(
        _g0_body,
        grid=(N // _NB,),
        in_specs=[_rows((_NB, D)), _full((D, D)), _full((D, D))],
        out_specs=(_rows((_NB, D)), _rows((_NB, D))),
        out_shape=(_sds((N, D)), _sds((N, D))),
    )(x, wa, wb)


def _make_edge_body(with_h):
    def body(*refs):
        if with_h:
            (gdg, gsg, ea, posd, poss, w1c, b1, w2c, bc, vrow, a2b,
             h_ref, a16_ref) = refs
        else:
            (gdg, gsg, ea, posd, poss, w1c, b1, w2c, bc, vrow, a2b,
             a16_ref) = refs
        pre = (gdg[...] + gsg[...]
               + jnp.dot(ea[...], w1c[...], preferred_element_type=_f32)
               + b1[...])
        h = jnp.maximum(pre, 0.0)
        if with_h:
            h_ref[...] = h
        t = jnp.maximum(
            jnp.dot(h.astype(_bf16), w2c[...],
                    preferred_element_type=_f32) + bc[...], 0.0)
        wv = jnp.sum(t * vrow[...], axis=1, keepdims=True) + a2b[0, 0]
        d16 = poss[...] - posd[...]
        r2 = jnp.sum(d16 * d16, axis=1, keepdims=True)
        rinv = lax.rsqrt(jnp.maximum(r2, 1e-12))
        cols = lax.broadcasted_iota(_i32, (1, 16), 1)
        a16_ref[...] = d16 * (wv * rinv) + (cols == 3).astype(_f32)
    return body


def _edge(gdg, gsg, ea, posd, poss, w1c, b1, w2c, bc, vrow, a2b, with_h):
    in_specs = [
        _rows((_EB, D)), _rows((_EB, D)), _rows((_EB, D)),
        _rows((_EB, 16)), _rows((_EB, 16)),
        _full((D, D)), _full((1, D)), _full((D, 2 * D)), _full((1, 2 * D)),
        _full((1, 2 * D)), _full((1, 8)),
    ]
    if with_h:
        out_specs = (_rows((_EB, D)), _rows((_EB, 16)))
        out_shape = (_sds((E, D)), _sds((E, 16)))
    else:
        out_specs = (_rows((_EB, 16)),)
        out_shape = (_sds((E, 16)),)
    res = pl.pallas_call(
        _make_edge_body(with_h),
        grid=(E // _EB,),
        in_specs=in_specs,
        out_specs=out_specs,
        out_shape=out_shape,
    )(gdg, gsg, ea, posd, poss, w1c, b1, w2c, bc, vrow, a2b)
    return res if with_h else res[0]


def _node_body(x_ref, sh0, sh1, sa0, sa1, posp,
               n1a, w2n1b, cvec, nb1, n2, nb2, wan, wbn,
               xn_ref, gd_ref, gs_ref, pospn_ref):
    sa = sa0[...] + sa1[...]
    counts = sa[:, 3:4]
    inv = 1.0 / jnp.maximum(counts, 1.0)
    flag = jnp.minimum(counts, 1.0)
    cols = lax.broadcasted_iota(_i32, (1, 16), 1)
    mask = (cols < 3).astype(_f32)
    pospn_ref[...] = posp[...] + sa * inv * mask
    sh = sh0[...] + sh1[...]
    z = (jnp.dot(x_ref[...], n1a[...], preferred_element_type=_f32)
         + jnp.dot(sh, w2n1b[...], preferred_element_type=_f32) * inv
         + flag * cvec[...] + nb1[...])
    xn = jnp.dot(jnp.maximum(z, 0.0), n2[...],
                 preferred_element_type=_f32) + nb2[...]
    xn_ref[...] = xn
    gd_ref[...] = jnp.dot(xn, wan[...], preferred_element_type=_f32)
    gs_ref[...] = jnp.dot(xn, wbn[...], preferred_element_type=_f32)


def _node(x, sh0, sh1, sa0, sa1, posp, n1a, w2n1b, cvec, nb1, n2, nb2,
          wan, wbn):
    return pl.pallas_call(
        _node_body,
        grid=(N // _NB,),
        in_specs=[
            _rows((_NB, D)), _rows((_NB, D)), _rows((_NB, D)),
            _rows((_NB, 16)), _rows((_NB, 16)), _rows((_NB, 16)),
            _full((D, 2 * D)), _full((D, 2 * D)), _full((1, 2 * D)),
            _full((1, 2 * D)), _full((2 * D, D)), _full((1, D)),
            _full((D, D)), _full((D, D)),
        ],
        out_specs=(_rows((_NB, D)), _rows((_NB, D)), _rows((_NB, D)),
                   _rows((_NB, 16))),
        out_shape=(_sds((N, D)), _sds((N, D)), _sds((N, D)), _sds((N, 16))),
    )(x, sh0, sh1, sa0, sa1, posp, n1a, w2n1b, cvec, nb1, n2, nb2, wan, wbn)


def _posup_body(sa0, sa1, posp, pospn_ref):
    sa = sa0[...] + sa1[...]
    counts = sa[:, 3:4]
    inv = 1.0 / jnp.maximum(counts, 1.0)
    cols = lax.broadcasted_iota(_i32, (1, 16), 1)
    mask = (cols < 3).astype(_f32)
    pospn_ref[...] = posp[...] + sa * inv * mask


def _posup(sa0, sa1, posp):
    return pl.pallas_call(
        _posup_body,
        grid=(N // _NB,),
        in_specs=[_rows((_NB, 16)), _rows((_NB, 16)), _rows((_NB, 16))],
        out_specs=_rows((_NB, 16)),
        out_shape=_sds((N, 16)),
    )(sa0, sa1, posp)


def kernel(x, edge_index, edge_attr, pos, params):
    srcv = edge_index[0].astype(_i32)
    dstv = edge_index[1].astype(_i32)
    posp = jnp.pad(pos, ((0, 0), (0, 13)))
    z128 = jnp.zeros((N, D), _f32)
    z16 = jnp.zeros((N, 16), _f32)

    folded = []
    for p in params:
        W1, b1 = p['msg']['W1'], p['msg']['b1']
        W2, b2 = p['msg']['W2'], p['msg']['b2']
        A1, a1b = p['accl']['W1'], p['accl']['b1']
        A2, a2b = p['accl']['W2'], p['accl']['b2']
        N1, nb1 = p['node']['W1'], p['node']['b1']
        N2, nb2 = p['node']['W2'], p['node']['b2']
        folded.append(dict(
            W1a=W1[:D], W1b=W1[D:2 * D], W1c=W1[2 * D:].astype(_bf16),
            b1=b1[None, :],
            W2c=(W2 @ A1).astype(_bf16), bc=(b2 @ A1 + a1b)[None, :],
            vrow=A2[:, 0][None, :],
            a2b=jnp.broadcast_to(a2b[0], (1, 8)),
            N1a=N1[:D], W2N1b=W2 @ N1[D:2 * D],
            cvec=(b2 @ N1[D:2 * D])[None, :], nb1=nb1[None, :],
            N2=N2, nb2=nb2[None, :],
        ))

    ea_bf = edge_attr.astype(_bf16)
    gd, gs = _g0(x, folded[0]['W1a'], folded[0]['W1b'])
    for l in range(3):
        f = folded[l]
        gdg, gsg, posd, poss = _assemble(gd, gs, posp, srcv, dstv)
        if l < 2:
            h, a16 = _edge(gdg, gsg, ea_bf, posd, poss, f['W1c'],
                           f['b1'], f['W2c'], f['bc'], f['vrow'], f['a2b'],
                           True)
            sh0, sh1, sa0, sa1 = _scatter_full(h, a16, dstv, z128, z16)
            fn = folded[l + 1]
            x, gd, gs, posp = _node(
                x, sh0, sh1, sa0, sa1, posp,
                f['N1a'], f['W2N1b'], f['cvec'], f['nb1'], f['N2'], f['nb2'],
                fn['W1a'], fn['W1b'])
        else:
            a16 = _edge(gdg, gsg, ea_bf, posd, poss, f['W1c'], f['b1'],
                        f['W2c'], f['bc'], f['vrow'], f['a2b'], False)
            sa0, sa1 = _scatter_aonly(a16, dstv, z16)
            posp = _posup(sa0, sa1, posp)
    return posp[:, :3]


# speedup vs baseline: 1.6000x; 1.0639x over previous
import functools

import jax
import jax.numpy as jnp
from jax import lax
from jax.experimental import pallas as pl
from jax.experimental.pallas import tpu as pltpu
from jax.experimental.pallas import tpu_sc as plsc

N = 10000
E = 160000
D = 128

NC = 2
NS = 16
NW = NC * NS
CHUNK = 128
NCHUNKS = E // CHUNK
GROUPS = CHUNK // 16
ROWS_A = 632
ROWS_LAST = N - 15 * ROWS_A

_MESH = plsc.VectorSubcoreMesh(core_axis_name="c", subcore_axis_name="s")
_SC_PARAMS = pltpu.CompilerParams(use_tc_tiling_on_sc=False)

_f32 = jnp.float32
_i32 = jnp.int32
_bf16 = jnp.bfloat16


def _sds(shape, dtype=_f32):
    return jax.ShapeDtypeStruct(shape, dtype)


_VCH = (NCHUNKS + NW - 1) // NW


@functools.partial(
    pl.kernel,
    out_type=(_sds((E, D)), _sds((E, D)), _sds((E, 16)), _sds((E, 16))),
    mesh=_MESH,
    scratch_types=[
        [pltpu.VMEM((CHUNK,), _i32)] * 2,
        [pltpu.VMEM((CHUNK,), _i32)] * 2,
        [pltpu.VMEM((CHUNK, D), _f32)] * 2,
        [pltpu.VMEM((CHUNK, D), _f32)] * 2,
        [pltpu.VMEM((CHUNK, 16), _f32)] * 2,
        [pltpu.VMEM((CHUNK, 16), _f32)] * 2,
        [pltpu.SemaphoreType.DMA] * 2,
        [pltpu.SemaphoreType.DMA] * 2,
        [pltpu.SemaphoreType.DMA] * 2,
    ],
    compiler_params=_SC_PARAMS,
)
def _assemble(gd_hbm, gs_hbm, p16_hbm, src_hbm, dst_hbm,
              gdg_hbm, gsg_hbm, posd_hbm, poss_hbm,
              idxd_v, idxs_v, bufa, bufb, bufpd, bufps, semi, semg, semw):
    wid = lax.axis_index("s") * NC + lax.axis_index("c")
    nch = (NCHUNKS - wid + NW - 1) // NW

    def fire_idx(j, p):
        base = (wid + j * NW) * CHUNK
        pltpu.async_copy(dst_hbm.at[pl.ds(base, CHUNK)], idxd_v[p], semi[p])
        pltpu.async_copy(src_hbm.at[pl.ds(base, CHUNK)], idxs_v[p], semi[p])

    def wait_idx(p):
        pltpu.make_async_copy(dst_hbm.at[pl.ds(0, CHUNK)], idxd_v[p],
                              semi[p]).wait()
        pltpu.make_async_copy(src_hbm.at[pl.ds(0, CHUNK)], idxs_v[p],
                              semi[p]).wait()

    def drain_writes(p):
        z = pl.ds(0, CHUNK)
        pltpu.make_async_copy(bufa[p], gdg_hbm.at[z], semw[p]).wait()
        pltpu.make_async_copy(bufb[p], gsg_hbm.at[z], semw[p]).wait()
        pltpu.make_async_copy(bufpd[p], posd_hbm.at[z], semw[p]).wait()
        pltpu.make_async_copy(bufps[p], poss_hbm.at[z], semw[p]).wait()

    def gathers_and_writes(j, p, fire_next_idx):
        cps = [
            pltpu.async_copy(gd_hbm.at[idxd_v[p]], bufa[p], semg[p]),
            pltpu.async_copy(gs_hbm.at[idxs_v[p]], bufb[p], semg[p]),
            pltpu.async_copy(p16_hbm.at[idxd_v[p]], bufpd[p], semg[p]),
            pltpu.async_copy(p16_hbm.at[idxs_v[p]], bufps[p], semg[p]),
        ]
        if fire_next_idx:
            fire_next_idx()
        for cp in cps:
            cp.wait()
        base = (wid + j * NW) * CHUNK
        pltpu.async_copy(bufa[p], gdg_hbm.at[pl.ds(base, CHUNK)], semw[p])
        pltpu.async_copy(bufb[p], gsg_hbm.at[pl.ds(base, CHUNK)], semw[p])
        pltpu.async_copy(bufpd[p], posd_hbm.at[pl.ds(base, CHUNK)], semw[p])
        pltpu.async_copy(bufps[p], poss_hbm.at[pl.ds(base, CHUNK)], semw[p])

    fire_idx(0, 0)

    def pair(i2, carry):
        for b in range(2):
            j = 2 * i2 + b
            p = b

            @pl.when(i2 >= 1)
            def _():
                drain_writes(p)

            wait_idx(p)
            gathers_and_writes(j, p, lambda: fire_idx(j + 1, 1 - p))
        return carry

    lax.fori_loop(0, (_VCH - 2) // 2, pair, 0)

    drain_writes(0)
    wait_idx(0)
    has39 = nch == _VCH

    def fire39():
        @pl.when(has39)
        def _():
            fire_idx(_VCH - 1, 1)

    gathers_and_writes(_VCH - 2, 0, fire39)
    drain_writes(1)

    @pl.when(has39)
    def _():
        wait_idx(1)
        gathers_and_writes(_VCH - 1, 1, None)
        drain_writes(1)

    drain_writes(0)


def _make_scatter(with_h):
    outs = []
    if with_h:
        outs += [_sds((N, D)), _sds((N, D))]
    outs += [_sds((N, 16)), _sds((N, 16))]

    scratch = [
        [pltpu.VMEM((CHUNK,), _i32)] * 2,
        [pltpu.VMEM((CHUNK, 16), _f32)] * 2,
        [pltpu.SemaphoreType.DMA] * 2,
    ]
    if with_h:
        scratch.insert(2, [pltpu.VMEM((CHUNK, D), _f32)] * 2)
        scratch += [pltpu.VMEM_SHARED((N, D), _f32)]
    scratch += [pltpu.VMEM_SHARED((N, 16), _f32)]

    def body(*refs):
        if with_h:
            (h_hbm, a_hbm, dst_hbm, z128_hbm, z16_hbm,
             sh0_hbm, sh1_hbm, sa0_hbm, sa1_hbm,
             idxd_v, abuf, hbuf,
             semin, sh_sp, sa_sp) = refs
        else:
            (a_hbm, dst_hbm, z16_hbm,
             sa0_hbm, sa1_hbm,
             idxd_v, abuf,
             semin, sa_sp) = refs

        cid = lax.axis_index("c")
        sid = lax.axis_index("s")
        wid = sid * NC + cid
        row0 = sid * ROWS_A

        @pl.when(sid < NS - 1)
        def _():
            pltpu.sync_copy(z16_hbm.at[pl.ds(row0, ROWS_A)],
                            sa_sp.at[pl.ds(row0, ROWS_A)])
            if with_h:
                pltpu.sync_copy(z128_hbm.at[pl.ds(row0, ROWS_A)],
                                sh_sp.at[pl.ds(row0, ROWS_A)])

        @pl.when(sid == NS - 1)
        def _():
            pltpu.sync_copy(z16_hbm.at[pl.ds(row0, ROWS_LAST)],
                            sa_sp.at[pl.ds(row0, ROWS_LAST)])
            if with_h:
                pltpu.sync_copy(z128_hbm.at[pl.ds(row0, ROWS_LAST)],
                                sh_sp.at[pl.ds(row0, ROWS_LAST)])
        plsc.subcore_barrier()

        nch = (NCHUNKS - wid + NW - 1) // NW

        def fire_in(j, p):
            base = (wid + j * NW) * CHUNK
            pltpu.async_copy(dst_hbm.at[pl.ds(base, CHUNK)], idxd_v[p],
                             semin[p])
            pltpu.async_copy(a_hbm.at[pl.ds(base, CHUNK)], abuf[p], semin[p])
            if with_h:
                pltpu.async_copy(h_hbm.at[pl.ds(base, CHUNK)], hbuf[p],
                                 semin[p])

        def wait_in(p):
            z = pl.ds(0, CHUNK)
            pltpu.make_async_copy(dst_hbm.at[z], idxd_v[p], semin[p]).wait()
            pltpu.make_async_copy(a_hbm.at[z], abuf[p], semin[p]).wait()
            if with_h:
                pltpu.make_async_copy(h_hbm.at[z], hbuf[p], semin[p]).wait()

        def scatter(p):
            pltpu.sync_copy(abuf[p], sa_sp.at[idxd_v[p]], add=True)
            if with_h:
                pltpu.sync_copy(hbuf[p], sh_sp.at[idxd_v[p]], add=True)

        fire_in(0, 0)

        def pair(i2, carry):
            for b in range(2):
                j = 2 * i2 + b
                wait_in(b)
                fire_in(j + 1, 1 - b)
                scatter(b)
            return carry

        lax.fori_loop(0, (_VCH - 2) // 2, pair, 0)
        has39 = nch == _VCH
        wait_in(0)

        @pl.when(has39)
        def _():
            fire_in(_VCH - 1, 1)

        scatter(0)

        @pl.when(has39)
        def _():
            wait_in(1)
            scatter(1)

        plsc.subcore_barrier()

        for nrows, is_last in ((ROWS_A, False), (ROWS_LAST, True)):
            cond = (sid == NS - 1) if is_last else (sid < NS - 1)
            rows = pl.ds(row0, nrows)

            @pl.when(jnp.logical_and(cond, cid == 0))
            def _(rows=rows):
                pltpu.sync_copy(sa_sp.at[rows], sa0_hbm.at[rows])
                if with_h:
                    pltpu.sync_copy(sh_sp.at[rows], sh0_hbm.at[rows])

            @pl.when(jnp.logical_and(cond, cid == 1))
            def _(rows=rows):
                pltpu.sync_copy(sa_sp.at[rows], sa1_hbm.at[rows])
                if with_h:
                    pltpu.sync_copy(sh_sp.at[rows], sh1_hbm.at[rows])

    return pl.kernel(body, out_type=tuple(outs), mesh=_MESH,
                     scratch_types=scratch, compiler_params=_SC_PARAMS)


_scatter_full = _make_scatter(True)
_scatter_aonly = _make_scatter(False)


_NB = 1000
_EB = 3200


def _full(shape):
    return pl.BlockSpec(shape, lambda i: (0,) * len(shape))


def _rows(shape):
    return pl.BlockSpec(shape, lambda i: (i,) + (0,) * (len(shape) - 1))


def _g0_body(x_ref, wa_ref, wb_ref, gd_ref, gs_ref):
    xb = x_ref[...]
    gd_ref[...] = jnp.dot(xb, wa_ref[...], preferred_element_type=_f32)
    gs_ref[...] = jnp.dot(xb, wb_ref[...], preferred_element_type=_f32)


def _g0(x, wa, wb):
    return pl.pallas_call(
        _g0_body,
        grid=(N // _NB,),
        in_specs=[_rows((_NB, D)), _full((D, D)), _full((D, D))],
        out_specs=(_rows((_NB, D)), _rows((_NB, D))),
        out_shape=(_sds((N, D)), _sds((N, D))),
    )(x, wa, wb)


def _make_edge_body(with_h):
    def body(*refs):
        if with_h:
            (gdg, gsg, ea, posd, poss, w1c, b1, w2c, bc, vrow, a2b,
             h_ref, a16_ref) = refs
        else:
            (gdg, gsg, ea, posd, poss, w1c, b1, w2c, bc, vrow, a2b,
             a16_ref) = refs
        pre = (gdg[...] + gsg[...]
               + jnp.dot(ea[...], w1c[...], preferred_element_type=_f32)
               + b1[...])
        h = jnp.maximum(pre, 0.0)
        if with_h:
            h_ref[...] = h
        t = jnp.maximum(
            jnp.dot(h.astype(_bf16), w2c[...],
                    preferred_element_type=_f32) + bc[...], 0.0)
        wv = jnp.sum(t * vrow[...], axis=1, keepdims=True) + a2b[0, 0]
        d16 = poss[...] - posd[...]
        r2 = jnp.sum(d16 * d16, axis=1, keepdims=True)
        rinv = lax.rsqrt(jnp.maximum(r2, 1e-12))
        cols = lax.broadcasted_iota(_i32, (1, 16), 1)
        a16_ref[...] = d16 * (wv * rinv) + (cols == 3).astype(_f32)
    return body


def _edge(gdg, gsg, ea, posd, poss, w1c, b1, w2c, bc, vrow, a2b, with_h):
    in_specs = [
        _rows((_EB, D)), _rows((_EB, D)), _rows((_EB, D)),
        _rows((_EB, 16)), _rows((_EB, 16)),
        _full((D, D)), _full((1, D)), _full((D, 2 * D)), _full((1, 2 * D)),
        _full((1, 2 * D)), _full((1, 8)),
    ]
    if with_h:
        out_specs = (_rows((_EB, D)), _rows((_EB, 16)))
        out_shape = (_sds((E, D)), _sds((E, 16)))
    else:
        out_specs = (_rows((_EB, 16)),)
        out_shape = (_sds((E, 16)),)
    res = pl.pallas_call(
        _make_edge_body(with_h),
        grid=(E // _EB,),
        in_specs=in_specs,
        out_specs=out_specs,
        out_shape=out_shape,
    )(gdg, gsg, ea, posd, poss, w1c, b1, w2c, bc, vrow, a2b)
    return res if with_h else res[0]


def _node_body(x_ref, sh0, sh1, sa0, sa1, posp,
               n1a, w2n1b, cvec, nb1, n2, nb2, wan, wbn,
               xn_ref, gd_ref, gs_ref, pospn_ref):
    sa = sa0[...] + sa1[...]
    counts = sa[:, 3:4]
    inv = 1.0 / jnp.maximum(counts, 1.0)
    flag = jnp.minimum(counts, 1.0)
    cols = lax.broadcasted_iota(_i32, (1, 16), 1)
    mask = (cols < 3).astype(_f32)
    pospn_ref[...] = posp[...] + sa * inv * mask
    sh = sh0[...] + sh1[...]
    z = (jnp.dot(x_ref[...], n1a[...], preferred_element_type=_f32)
         + jnp.dot(sh, w2n1b[...], preferred_element_type=_f32) * inv
         + flag * cvec[...] + nb1[...])
    xn = jnp.dot(jnp.maximum(z, 0.0), n2[...],
                 preferred_element_type=_f32) + nb2[...]
    xn_ref[...] = xn
    gd_ref[...] = jnp.dot(xn, wan[...], preferred_element_type=_f32)
    gs_ref[...] = jnp.dot(xn, wbn[...], preferred_element_type=_f32)


def _node(x, sh0, sh1, sa0, sa1, posp, n1a, w2n1b, cvec, nb1, n2, nb2,
          wan, wbn):
    return pl.pallas_call(
        _node_body,
        grid=(N // _NB,),
        in_specs=[
            _rows((_NB, D)), _rows((_NB, D)), _rows((_NB, D)),
            _rows((_NB, 16)), _rows((_NB, 16)), _rows((_NB, 16)),
            _full((D, 2 * D)), _full((D, 2 * D)), _full((1, 2 * D)),
            _full((1, 2 * D)), _full((2 * D, D)), _full((1, D)),
            _full((D, D)), _full((D, D)),
        ],
        out_specs=(_rows((_NB, D)), _rows((_NB, D)), _rows((_NB, D)),
                   _rows((_NB, 16))),
        out_shape=(_sds((N, D)), _sds((N, D)), _sds((N, D)), _sds((N, 16))),
    )(x, sh0, sh1, sa0, sa1, posp, n1a, w2n1b, cvec, nb1, n2, nb2, wan, wbn)


def _posup_body(sa0, sa1, posp, pospn_ref):
    sa = sa0[...] + sa1[...]
    counts = sa[:, 3:4]
    inv = 1.0 / jnp.maximum(counts, 1.0)
    cols = lax.broadcasted_iota(_i32, (1, 16), 1)
    mask = (cols < 3).astype(_f32)
    pospn_ref[...] = posp[...] + sa * inv * mask


def _posup(sa0, sa1, posp):
    return pl.pallas_call(
        _posup_body,
        grid=(N // _NB,),
        in_specs=[_rows((_NB, 16)), _rows((_NB, 16)), _rows((_NB, 16))],
        out_specs=_rows((_NB, 16)),
        out_shape=_sds((N, 16)),
    )(sa0, sa1, posp)


def kernel(x, edge_index, edge_attr, pos, params):
    srcv = edge_index[0].astype(_i32)
    dstv = edge_index[1].astype(_i32)
    posp = jnp.pad(pos, ((0, 0), (0, 13)))
    z128 = jnp.zeros((N, D), _f32)
    z16 = jnp.zeros((N, 16), _f32)

    folded = []
    for p in params:
        W1, b1 = p['msg']['W1'], p['msg']['b1']
        W2, b2 = p['msg']['W2'], p['msg']['b2']
        A1, a1b = p['accl']['W1'], p['accl']['b1']
        A2, a2b = p['accl']['W2'], p['accl']['b2']
        N1, nb1 = p['node']['W1'], p['node']['b1']
        N2, nb2 = p['node']['W2'], p['node']['b2']
        folded.append(dict(
            W1a=W1[:D], W1b=W1[D:2 * D], W1c=W1[2 * D:].astype(_bf16),
            b1=b1[None, :],
            W2c=(W2 @ A1).astype(_bf16), bc=(b2 @ A1 + a1b)[None, :],
            vrow=A2[:, 0][None, :],
            a2b=jnp.broadcast_to(a2b[0], (1, 8)),
            N1a=N1[:D], W2N1b=W2 @ N1[D:2 * D],
            cvec=(b2 @ N1[D:2 * D])[None, :], nb1=nb1[None, :],
            N2=N2, nb2=nb2[None, :],
        ))

    ea_bf = edge_attr.astype(_bf16)
    gd, gs = _g0(x, folded[0]['W1a'], folded[0]['W1b'])
    for l in range(3):
        f = folded[l]
        gdg, gsg, posd, poss = _assemble(gd, gs, posp, srcv, dstv)
        if l < 2:
            h, a16 = _edge(gdg, gsg, ea_bf, posd, poss, f['W1c'],
                           f['b1'], f['W2c'], f['bc'], f['vrow'], f['a2b'],
                           True)
            sh0, sh1, sa0, sa1 = _scatter_full(h, a16, dstv, z128, z16)
            fn = folded[l + 1]
            x, gd, gs, posp = _node(
                x, sh0, sh1, sa0, sa1, posp,
                f['N1a'], f['W2N1b'], f['cvec'], f['nb1'], f['N2'], f['nb2'],
                fn['W1a'], fn['W1b'])
        else:
            a16 = _edge(gdg, gsg, ea_bf, posd, poss, f['W1c'], f['b1'],
                        f['W2c'], f['bc'], f['vrow'], f['a2b'], False)
            sa0, sa1 = _scatter_aonly(a16, dstv, z16)
            posp = _posup(sa0, sa1, posp)
    return posp[:, :3]


# speedup vs baseline: 1.6012x; 1.0008x over previous
import functools

import jax
import jax.numpy as jnp
from jax import lax
from jax.experimental import pallas as pl
from jax.experimental.pallas import tpu as pltpu
from jax.experimental.pallas import tpu_sc as plsc

N = 10000
E = 160000
D = 128

NC = 2
NS = 16
NW = NC * NS
CHUNK = 128
NCHUNKS = E // CHUNK
GROUPS = CHUNK // 16
ROWS_A = 632
ROWS_LAST = N - 15 * ROWS_A

_MESH = plsc.VectorSubcoreMesh(core_axis_name="c", subcore_axis_name="s")
_SC_PARAMS = pltpu.CompilerParams(use_tc_tiling_on_sc=False)

_f32 = jnp.float32
_i32 = jnp.int32
_bf16 = jnp.bfloat16


def _sds(shape, dtype=_f32):
    return jax.ShapeDtypeStruct(shape, dtype)


_VCH = (NCHUNKS + NW - 1) // NW


@functools.partial(
    pl.kernel,
    out_type=(_sds((NCHUNKS, CHUNK, D)), _sds((NCHUNKS, CHUNK, D)),
              _sds((E, 16)), _sds((E, 16))),
    mesh=_MESH,
    scratch_types=[
        [pltpu.VMEM((CHUNK,), _i32)] * 2,
        [pltpu.VMEM((CHUNK,), _i32)] * 2,
        [pltpu.VMEM((CHUNK, D), _f32)] * 2,
        [pltpu.VMEM((CHUNK, D), _f32)] * 2,
        [pltpu.VMEM((CHUNK, 16), _f32)] * 2,
        [pltpu.VMEM((CHUNK, 16), _f32)] * 2,
        [pltpu.SemaphoreType.DMA] * 2,
        [pltpu.SemaphoreType.DMA] * 2,
        [pltpu.SemaphoreType.DMA] * 2,
    ],
    compiler_params=_SC_PARAMS,
)
def _assemble(gd_hbm, gs_hbm, p16_hbm, src_hbm, dst_hbm,
              gdg_hbm, gsg_hbm, posd_hbm, poss_hbm,
              idxd_v, idxs_v, bufa, bufb, bufpd, bufps, semi, semg, semw):
    wid = lax.axis_index("s") * NC + lax.axis_index("c")
    nch = (NCHUNKS - wid + NW - 1) // NW

    def fire_idx(j, p):
        base = (wid + j * NW) * CHUNK
        pltpu.async_copy(dst_hbm.at[pl.ds(base, CHUNK)], idxd_v[p], semi[p])
        pltpu.async_copy(src_hbm.at[pl.ds(base, CHUNK)], idxs_v[p], semi[p])

    def wait_idx(p):
        pltpu.make_async_copy(dst_hbm.at[pl.ds(0, CHUNK)], idxd_v[p],
                              semi[p]).wait()
        pltpu.make_async_copy(src_hbm.at[pl.ds(0, CHUNK)], idxs_v[p],
                              semi[p]).wait()

    def drain_writes(p):
        z = pl.ds(0, CHUNK)
        pltpu.make_async_copy(bufa[p], gdg_hbm.at[0], semw[p]).wait()
        pltpu.make_async_copy(bufb[p], gsg_hbm.at[0], semw[p]).wait()
        pltpu.make_async_copy(bufpd[p], posd_hbm.at[z], semw[p]).wait()
        pltpu.make_async_copy(bufps[p], poss_hbm.at[z], semw[p]).wait()

    def gathers_and_writes(j, p, fire_next_idx):
        cps = [
            pltpu.async_copy(gd_hbm.at[idxd_v[p]], bufa[p], semg[p]),
            pltpu.async_copy(gs_hbm.at[idxs_v[p]], bufb[p], semg[p]),
            pltpu.async_copy(p16_hbm.at[idxd_v[p]], bufpd[p], semg[p]),
            pltpu.async_copy(p16_hbm.at[idxs_v[p]], bufps[p], semg[p]),
        ]
        if fire_next_idx:
            fire_next_idx()
        for cp in cps:
            cp.wait()
        c = wid + j * NW
        base = c * CHUNK
        pltpu.async_copy(bufa[p], gdg_hbm.at[c], semw[p])
        pltpu.async_copy(bufb[p], gsg_hbm.at[c], semw[p])
        pltpu.async_copy(bufpd[p], posd_hbm.at[pl.ds(base, CHUNK)], semw[p])
        pltpu.async_copy(bufps[p], poss_hbm.at[pl.ds(base, CHUNK)], semw[p])

    fire_idx(0, 0)

    def pair(i2, carry):
        for b in range(2):
            j = 2 * i2 + b
            p = b

            @pl.when(i2 >= 1)
            def _():
                drain_writes(p)

            wait_idx(p)
            gathers_and_writes(j, p, lambda: fire_idx(j + 1, 1 - p))
        return carry

    lax.fori_loop(0, (_VCH - 2) // 2, pair, 0)

    drain_writes(0)
    wait_idx(0)
    has39 = nch == _VCH

    def fire39():
        @pl.when(has39)
        def _():
            fire_idx(_VCH - 1, 1)

    gathers_and_writes(_VCH - 2, 0, fire39)
    drain_writes(1)

    @pl.when(has39)
    def _():
        wait_idx(1)
        gathers_and_writes(_VCH - 1, 1, None)
        drain_writes(1)

    drain_writes(0)


def _make_scatter(with_h):
    outs = []
    if with_h:
        outs += [_sds((N, D)), _sds((N, D))]
    outs += [_sds((N, 16)), _sds((N, 16))]

    scratch = [
        [pltpu.VMEM((CHUNK,), _i32)] * 2,
        [pltpu.VMEM((CHUNK, 16), _f32)] * 2,
        [pltpu.SemaphoreType.DMA] * 2,
    ]
    if with_h:
        scratch.insert(2, [pltpu.VMEM((CHUNK, D), _f32)] * 2)
        scratch += [pltpu.VMEM_SHARED((N, D), _f32)]
    scratch += [pltpu.VMEM_SHARED((N, 16), _f32)]

    def body(*refs):
        if with_h:
            (h_hbm, a_hbm, dst_hbm, z128_hbm, z16_hbm,
             sh0_hbm, sh1_hbm, sa0_hbm, sa1_hbm,
             idxd_v, abuf, hbuf,
             semin, sh_sp, sa_sp) = refs
        else:
            (a_hbm, dst_hbm, z16_hbm,
             sa0_hbm, sa1_hbm,
             idxd_v, abuf,
             semin, sa_sp) = refs

        cid = lax.axis_index("c")
        sid = lax.axis_index("s")
        wid = sid * NC + cid
        row0 = sid * ROWS_A

        @pl.when(sid < NS - 1)
        def _():
            pltpu.sync_copy(z16_hbm.at[pl.ds(row0, ROWS_A)],
                            sa_sp.at[pl.ds(row0, ROWS_A)])
            if with_h:
                pltpu.sync_copy(z128_hbm.at[pl.ds(row0, ROWS_A)],
                                sh_sp.at[pl.ds(row0, ROWS_A)])

        @pl.when(sid == NS - 1)
        def _():
            pltpu.sync_copy(z16_hbm.at[pl.ds(row0, ROWS_LAST)],
                            sa_sp.at[pl.ds(row0, ROWS_LAST)])
            if with_h:
                pltpu.sync_copy(z128_hbm.at[pl.ds(row0, ROWS_LAST)],
                                sh_sp.at[pl.ds(row0, ROWS_LAST)])
        plsc.subcore_barrier()

        nch = (NCHUNKS - wid + NW - 1) // NW

        def fire_in(j, p):
            c = wid + j * NW
            base = c * CHUNK
            pltpu.async_copy(dst_hbm.at[pl.ds(base, CHUNK)], idxd_v[p],
                             semin[p])
            pltpu.async_copy(a_hbm.at[pl.ds(base, CHUNK)], abuf[p], semin[p])
            if with_h:
                pltpu.async_copy(h_hbm.at[c], hbuf[p], semin[p])

        def wait_in(p):
            z = pl.ds(0, CHUNK)
            pltpu.make_async_copy(dst_hbm.at[z], idxd_v[p], semin[p]).wait()
            pltpu.make_async_copy(a_hbm.at[z], abuf[p], semin[p]).wait()
            if with_h:
                pltpu.make_async_copy(h_hbm.at[0], hbuf[p], semin[p]).wait()

        def scatter(p):
            pltpu.sync_copy(abuf[p], sa_sp.at[idxd_v[p]], add=True)
            if with_h:
                pltpu.sync_copy(hbuf[p], sh_sp.at[idxd_v[p]], add=True)

        fire_in(0, 0)

        def pair(i2, carry):
            for b in range(2):
                j = 2 * i2 + b
                wait_in(b)
                fire_in(j + 1, 1 - b)
                scatter(b)
            return carry

        lax.fori_loop(0, (_VCH - 2) // 2, pair, 0)
        has39 = nch == _VCH
        wait_in(0)

        @pl.when(has39)
        def _():
            fire_in(_VCH - 1, 1)

        scatter(0)

        @pl.when(has39)
        def _():
            wait_in(1)
            scatter(1)

        plsc.subcore_barrier()

        for nrows, is_last in ((ROWS_A, False), (ROWS_LAST, True)):
            cond = (sid == NS - 1) if is_last else (sid < NS - 1)
            rows = pl.ds(row0, nrows)

            @pl.when(jnp.logical_and(cond, cid == 0))
            def _(rows=rows):
                pltpu.sync_copy(sa_sp.at[rows], sa0_hbm.at[rows])
                if with_h:
                    pltpu.sync_copy(sh_sp.at[rows], sh0_hbm.at[rows])

            @pl.when(jnp.logical_and(cond, cid == 1))
            def _(rows=rows):
                pltpu.sync_copy(sa_sp.at[rows], sa1_hbm.at[rows])
                if with_h:
                    pltpu.sync_copy(sh_sp.at[rows], sh1_hbm.at[rows])

    return pl.kernel(body, out_type=tuple(outs), mesh=_MESH,
                     scratch_types=scratch, compiler_params=_SC_PARAMS)


_scatter_full = _make_scatter(True)
_scatter_aonly = _make_scatter(False)


_NB = 1000
_EB = 3200


def _full(shape):
    return pl.BlockSpec(shape, lambda i: (0,) * len(shape))


def _rows(shape):
    return pl.BlockSpec(shape, lambda i: (i,) + (0,) * (len(shape) - 1))


def _g0_body(x_ref, wa_ref, wb_ref, gd_ref, gs_ref):
    xb = x_ref[...]
    gd_ref[...] = jnp.dot(xb, wa_ref[...], preferred_element_type=_f32)
    gs_ref[...] = jnp.dot(xb, wb_ref[...], preferred_element_type=_f32)


def _g0(x, wa, wb):
    return pl.pallas_call(
        _g0_body,
        grid=(N // _NB,),
        in_specs=[_rows((_NB, D)), _full((D, D)), _full((D, D))],
        out_specs=(_rows((_NB, D)), _rows((_NB, D))),
        out_shape=(_sds((N, D)), _sds((N, D))),
    )(x, wa, wb)


def _make_edge_body(with_h):
    def body(*refs):
        if with_h:
            (gdg, gsg, ea, posd, poss, w1c, b1, w2c, bc, vrow, a2b,
             h_ref, a16_ref) = refs
        else:
            (gdg, gsg, ea, posd, poss, w1c, b1, w2c, bc, vrow, a2b,
             a16_ref) = refs
        gdg2 = gdg[...].reshape(_EB, D)
        gsg2 = gsg[...].reshape(_EB, D)
        pre = (gdg2 + gsg2
               + jnp.dot(ea[...], w1c[...], preferred_element_type=_f32)
               + b1[...])
        h = jnp.maximum(pre, 0.0)
        if with_h:
            h_ref[...] = h.reshape(_EB // CHUNK, CHUNK, D)
        t = jnp.maximum(
            jnp.dot(h.astype(_bf16), w2c[...],
                    preferred_element_type=_f32) + bc[...], 0.0)
        wv = jnp.sum(t * vrow[...], axis=1, keepdims=True) + a2b[0, 0]
        d16 = poss[...] - posd[...]
        r2 = jnp.sum(d16 * d16, axis=1, keepdims=True)
        rinv = lax.rsqrt(jnp.maximum(r2, 1e-12))
        cols = lax.broadcasted_iota(_i32, (1, 16), 1)
        a16_ref[...] = d16 * (wv * rinv) + (cols == 3).astype(_f32)
    return body


def _edge(gdg, gsg, ea, posd, poss, w1c, b1, w2c, bc, vrow, a2b, with_h):
    ch3 = _rows((_EB // CHUNK, CHUNK, D))
    in_specs = [
        ch3, ch3, _rows((_EB, D)),
        _rows((_EB, 16)), _rows((_EB, 16)),
        _full((D, D)), _full((1, D)), _full((D, 2 * D)), _full((1, 2 * D)),
        _full((1, 2 * D)), _full((1, 8)),
    ]
    if with_h:
        out_specs = (ch3, _rows((_EB, 16)))
        out_shape = (_sds((NCHUNKS, CHUNK, D)), _sds((E, 16)))
    else:
        out_specs = (_rows((_EB, 16)),)
        out_shape = (_sds((E, 16)),)
    res = pl.pallas_call(
        _make_edge_body(with_h),
        grid=(E // _EB,),
        in_specs=in_specs,
        out_specs=out_specs,
        out_shape=out_shape,
    )(gdg, gsg, ea, posd, poss, w1c, b1, w2c, bc, vrow, a2b)
    return res if with_h else res[0]


def _node_body(x_ref, sh0, sh1, sa0, sa1, posp,
               n1a, w2n1b, cvec, nb1, n2, nb2, wan, wbn,
               xn_ref, gd_ref, gs_ref, pospn_ref):
    sa = sa0[...] + sa1[...]
    counts = sa[:, 3:4]
    inv = 1.0 / jnp.maximum(counts, 1.0)
    flag = jnp.minimum(counts, 1.0)
    cols = lax.broadcasted_iota(_i32, (1, 16), 1)
    mask = (cols < 3).astype(_f32)
    pospn_ref[...] = posp[...] + sa * inv * mask
    sh = sh0[...] + sh1[...]
    z = (jnp.dot(x_ref[...], n1a[...], preferred_element_type=_f32)
         + jnp.dot(sh, w2n1b[...], preferred_element_type=_f32) * inv
         + flag * cvec[...] + nb1[...])
    xn = jnp.dot(jnp.maximum(z, 0.0), n2[...],
                 preferred_element_type=_f32) + nb2[...]
    xn_ref[...] = xn
    gd_ref[...] = jnp.dot(xn, wan[...], preferred_element_type=_f32)
    gs_ref[...] = jnp.dot(xn, wbn[...], preferred_element_type=_f32)


def _node(x, sh0, sh1, sa0, sa1, posp, n1a, w2n1b, cvec, nb1, n2, nb2,
          wan, wbn):
    return pl.pallas_call(
        _node_body,
        grid=(N // _NB,),
        in_specs=[
            _rows((_NB, D)), _rows((_NB, D)), _rows((_NB, D)),
            _rows((_NB, 16)), _rows((_NB, 16)), _rows((_NB, 16)),
            _full((D, 2 * D)), _full((D, 2 * D)), _full((1, 2 * D)),
            _full((1, 2 * D)), _full((2 * D, D)), _full((1, D)),
            _full((D, D)), _full((D, D)),
        ],
        out_specs=(_rows((_NB, D)), _rows((_NB, D)), _rows((_NB, D)),
                   _rows((_NB, 16))),
        out_shape=(_sds((N, D)), _sds((N, D)), _sds((N, D)), _sds((N, 16))),
    )(x, sh0, sh1, sa0, sa1, posp, n1a, w2n1b, cvec, nb1, n2, nb2, wan, wbn)


def _posup_body(sa0, sa1, posp, pospn_ref):
    sa = sa0[...] + sa1[...]
    counts = sa[:, 3:4]
    inv = 1.0 / jnp.maximum(counts, 1.0)
    cols = lax.broadcasted_iota(_i32, (1, 16), 1)
    mask = (cols < 3).astype(_f32)
    pospn_ref[...] = posp[...] + sa * inv * mask


def _posup(sa0, sa1, posp):
    return pl.pallas_call(
        _posup_body,
        grid=(N // _NB,),
        in_specs=[_rows((_NB, 16)), _rows((_NB, 16)), _rows((_NB, 16))],
        out_specs=_rows((_NB, 16)),
        out_shape=_sds((N, 16)),
    )(sa0, sa1, posp)


def kernel(x, edge_index, edge_attr, pos, params):
    srcv = edge_index[0].astype(_i32)
    dstv = edge_index[1].astype(_i32)
    posp = jnp.pad(pos, ((0, 0), (0, 13)))
    z128 = jnp.zeros((N, D), _f32)
    z16 = jnp.zeros((N, 16), _f32)

    folded = []
    for p in params:
        W1, b1 = p['msg']['W1'], p['msg']['b1']
        W2, b2 = p['msg']['W2'], p['msg']['b2']
        A1, a1b = p['accl']['W1'], p['accl']['b1']
        A2, a2b = p['accl']['W2'], p['accl']['b2']
        N1, nb1 = p['node']['W1'], p['node']['b1']
        N2, nb2 = p['node']['W2'], p['node']['b2']
        folded.append(dict(
            W1a=W1[:D], W1b=W1[D:2 * D], W1c=W1[2 * D:].astype(_bf16),
            b1=b1[None, :],
            W2c=(W2 @ A1).astype(_bf16), bc=(b2 @ A1 + a1b)[None, :],
            vrow=A2[:, 0][None, :],
            a2b=jnp.broadcast_to(a2b[0], (1, 8)),
            N1a=N1[:D], W2N1b=W2 @ N1[D:2 * D],
            cvec=(b2 @ N1[D:2 * D])[None, :], nb1=nb1[None, :],
            N2=N2, nb2=nb2[None, :],
        ))

    ea_bf = edge_attr.astype(_bf16)
    gd, gs = _g0(x, folded[0]['W1a'], folded[0]['W1b'])
    for l in range(3):
        f = folded[l]
        gdg, gsg, posd, poss = _assemble(gd, gs, posp, srcv, dstv)
        if l < 2:
            h, a16 = _edge(gdg, gsg, ea_bf, posd, poss, f['W1c'],
                           f['b1'], f['W2c'], f['bc'], f['vrow'], f['a2b'],
                           True)
            sh0, sh1, sa0, sa1 = _scatter_full(h, a16, dstv, z128, z16)
            fn = folded[l + 1]
            x, gd, gs, posp = _node(
                x, sh0, sh1, sa0, sa1, posp,
                f['N1a'], f['W2N1b'], f['cvec'], f['nb1'], f['N2'], f['nb2'],
                fn['W1a'], fn['W1b'])
        else:
            a16 = _edge(gdg, gsg, ea_bf, posd, poss, f['W1c'], f['b1'],
                        f['W2c'], f['bc'], f['vrow'], f['a2b'], False)
            sa0, sa1 = _scatter_aonly(a16, dstv, z16)
            posp = _posup(sa0, sa1, posp)
    return posp[:, :3]
